# Initial kernel scaffold; baseline (speedup 1.0000x reference)
import jax
import jax.numpy as jnp
from jax.experimental import pallas as pl


def kernel(x, edge_index, precomp, connection, params):
    raise NotImplementedError("write your pallas kernel here")


# speedup vs baseline: 7.7890x; 7.7890x over previous
from functools import partial

import jax
import jax.numpy as jnp
from jax.experimental import pallas as pl

_B = 5


def _rep_dim(order):
    return 1 if order == 0 else 2 * order + 1


def _bf(v):
    return v.astype(jnp.bfloat16).astype(jnp.float32)


def _edge_body(pre_ref, trig_ref, g_ref, rw_ref, m_ref, k_ref, out_ref, *,
               in_order, ri, ro, co, ci):
    pre = pre_ref[...]
    rw = rw_ref[...]
    coeff = (_bf(pre[:, :_B]) * _bf(rw[:, :_B])
             + _bf(pre[:, _B:]) * _bf(rw[:, _B:]))
    m2d = m_ref[...]
    T = jnp.dot(coeff.astype(jnp.bfloat16), m2d.astype(jnp.bfloat16),
                preferred_element_type=jnp.float32)
    g = g_ref[...]
    if in_order == 0:
        ys = [_bf(g)]
    else:
        trig = trig_ref[...]
        msgs = [g[:, :ci]]
        for m in range(1, in_order + 1):
            c = trig[:, 2 * m - 2:2 * m - 1]
            s = trig[:, 2 * m - 1:2 * m]
            re = g[:, (2 * m - 1) * ci:(2 * m) * ci]
            im = g[:, (2 * m) * ci:(2 * m + 1) * ci]
            msgs.append(c * re - s * im)
            msgs.append(s * re + c * im)
        kb = k_ref[...].astype(jnp.bfloat16)
        ys = [_bf(jnp.dot(m_.astype(jnp.bfloat16), kb,
                          preferred_element_type=jnp.float32)) for m_ in msgs]
    for s in range(ro):
        acc = None
        for r in range(ri):
            term = _bf(T[:, r * ro + s:r * ro + s + 1]) * ys[r]
            acc = term if acc is None else acc + term
        out_ref[:, s * co:(s + 1) * co] = acc


def _edge_math(g, pre2, trig, rw2, m2d, K, in_order, ro, co, blk=512):
    E = g.shape[0]
    ri = _rep_dim(in_order)
    ci = K.shape[0]
    return pl.pallas_call(
        partial(_edge_body, in_order=in_order, ri=ri, ro=ro, co=co, ci=ci),
        grid=(E // blk,),
        in_specs=[
            pl.BlockSpec((blk, 2 * _B), lambda i: (i, 0)),
            pl.BlockSpec((blk, 4), lambda i: (i, 0)),
            pl.BlockSpec((blk, g.shape[1]), lambda i: (i, 0)),
            pl.BlockSpec((1, 2 * _B), lambda i: (0, 0)),
            pl.BlockSpec((_B, ri * ro), lambda i: (0, 0)),
            pl.BlockSpec((ci, co), lambda i: (0, 0)),
        ],
        out_specs=pl.BlockSpec((blk, ro * co), lambda i: (i, 0)),
        out_shape=jax.ShapeDtypeStruct((E, ro * co), jnp.float32),
    )(pre2, trig, g, rw2, m2d, K)


def _conv(h, src, dst, pre2, trig, deg, rw, M, K, in_order):
    n = h.shape[0]
    ri = _rep_dim(in_order)
    ro, co = M.shape[2], K.shape[1]
    if in_order == 0:
        table = jnp.einsum('nri,io->nro', h, K).reshape(n, co)
    else:
        table = h.reshape(n, ri * K.shape[0])
    g = jnp.take(table, src, axis=0)
    out_msg = _edge_math(g, pre2, trig, rw.reshape(1, 2 * _B),
                         M.reshape(_B, ri * ro), K, in_order, ro, co)
    out = jnp.zeros((n, ro * co), jnp.float32).at[dst].add(out_msg)
    out = out / deg[:, None]
    return out.reshape(n, ro, co)


def _bn(h, g, b):
    mean = h.mean(axis=(0, 1), keepdims=True)
    var = h.var(axis=(0, 1), keepdims=True)
    return (h - mean) / jnp.sqrt(var + 1e-5) * g + b


def kernel(x, edge_index, precomp, connection, params):
    src, dst = edge_index[0], edge_index[1]
    n = x.shape[0]
    E = src.shape[0]
    pre2 = precomp.reshape(E, 2 * _B)
    trig = jnp.stack([jnp.cos(connection), jnp.sin(connection),
                      jnp.cos(2 * connection), jnp.sin(2 * connection)], axis=1)
    deg = jnp.clip(jnp.zeros((n,), jnp.float32).at[dst].add(1.0), 1.0)
    blocks = [(0, 2, 128, 64, False), (2, 2, 64, 64, False), (2, 0, 64, 10, True)]
    h = x[:, None, :]
    for i, (oi, oo, ci, co, last) in enumerate(blocks):
        def p(name, i=i):
            return params[f"b{i}_{name}"]
        t = _conv(h, src, dst, pre2, trig, deg, p('rw1'), p('M1'), p('K1'), oi)
        t = _bn(t, p('g1'), p('be1'))
        t = jax.nn.relu(t)
        t = _conv(t, src, dst, pre2, trig, deg, p('rw2'), p('M2'), p('K2'), oo)
        t = _bn(t, p('g2'), p('be2'))
        sc = jnp.einsum('nri,rs,io->nso', h, p('S'), p('Ks'))
        h = t + sc
        if not last:
            h = jax.nn.relu(h)
    return h[:, 0, :]


# speedup vs baseline: 8.7025x; 1.1173x over previous
from functools import partial

import jax
import jax.numpy as jnp
from jax import lax
from jax.experimental import pallas as pl
from jax.experimental.pallas import tpu as pltpu
from jax.experimental.pallas import tpu_sc as plsc

_B = 5
_CH = 80
_NS = 16


def _rep_dim(order):
    return 1 if order == 0 else 2 * order + 1


def _bf(v):
    return v.astype(jnp.bfloat16).astype(jnp.float32)


def _edge_body(pre_ref, trig_ref, g_ref, rw_ref, m_ref, k_ref, out_ref, *,
               in_order, ri, ro, co, ci, dpad):
    pre = pre_ref[...]
    rw = rw_ref[...]
    coeff = (_bf(pre[:, :_B]) * _bf(rw[:, :_B])
             + _bf(pre[:, _B:]) * _bf(rw[:, _B:]))
    m2d = m_ref[...]
    T = jnp.dot(coeff.astype(jnp.bfloat16), m2d.astype(jnp.bfloat16),
                preferred_element_type=jnp.float32)
    g = g_ref[...]
    if in_order == 0:
        ys = [_bf(g)]
    else:
        trig = trig_ref[...]
        msgs = [g[:, :ci]]
        for m in range(1, in_order + 1):
            c = trig[:, 2 * m - 2:2 * m - 1]
            s = trig[:, 2 * m - 1:2 * m]
            re = g[:, (2 * m - 1) * ci:(2 * m) * ci]
            im = g[:, (2 * m) * ci:(2 * m + 1) * ci]
            msgs.append(c * re - s * im)
            msgs.append(s * re + c * im)
        kb = k_ref[...].astype(jnp.bfloat16)
        ys = [_bf(jnp.dot(m_.astype(jnp.bfloat16), kb,
                          preferred_element_type=jnp.float32)) for m_ in msgs]
    for s in range(ro):
        acc = None
        for r in range(ri):
            term = _bf(T[:, r * ro + s:r * ro + s + 1]) * ys[r]
            acc = term if acc is None else acc + term
        out_ref[:, s * co:(s + 1) * co] = acc
    if dpad > ro * co:
        out_ref[:, ro * co:] = jnp.zeros((out_ref.shape[0], dpad - ro * co),
                                         jnp.float32)


def _edge_math(g, pre2, trig, rw2, m2d, K, in_order, ro, co, dpad, blk=512):
    E = g.shape[0]
    ri = _rep_dim(in_order)
    ci = K.shape[0]
    return pl.pallas_call(
        partial(_edge_body, in_order=in_order, ri=ri, ro=ro, co=co, ci=ci,
                dpad=dpad),
        grid=(E // blk,),
        in_specs=[
            pl.BlockSpec((blk, 2 * _B), lambda i: (i, 0)),
            pl.BlockSpec((blk, 4), lambda i: (i, 0)),
            pl.BlockSpec((blk, g.shape[1]), lambda i: (i, 0)),
            pl.BlockSpec((1, 2 * _B), lambda i: (0, 0)),
            pl.BlockSpec((_B, ri * ro), lambda i: (0, 0)),
            pl.BlockSpec((ci, co), lambda i: (0, 0)),
        ],
        out_specs=pl.BlockSpec((blk, dpad), lambda i: (i, 0)),
        out_shape=jax.ShapeDtypeStruct((E, dpad), jnp.float32),
    )(pre2, trig, g, rw2, m2d, K)


def _sc_scatter_add(vals, idx2d, zrows, n):
    E, D = vals.shape
    EP = E // _NS
    NCH = EP // _CH
    npad = _NS * ((n + _NS * _CH - 1) // (_NS * _CH)) * _CH
    NROW = npad // _NS
    G = D // 128
    group_offs = (tuple(range(0, (G + 1) // 2 * 128, 128)),
                  tuple(range((G + 1) // 2 * 128, D, 128)))

    @partial(pl.kernel,
             out_type=jax.ShapeDtypeStruct((npad, D), jnp.float32),
             mesh=plsc.VectorSubcoreMesh(core_axis_name="c",
                                         subcore_axis_name="s"),
             scratch_types=[
                 pltpu.VMEM_SHARED((npad, 128), jnp.float32),
                 pltpu.VMEM((NCH, 1, _CH), jnp.int32),
                 pltpu.VMEM((_CH, 128), jnp.float32),
             ])
    def scat(vals_hbm, idx_hbm, z_hbm, out_hbm, acc, idxv, buf):
        c = lax.axis_index("c")
        s = lax.axis_index("s")
        row0 = s * NROW
        pltpu.sync_copy(idx_hbm.at[pl.ds(s * NCH, NCH)], idxv)

        for core_id, offs in enumerate(group_offs):
            if not offs:
                continue

            @pl.when(c == core_id)
            def _(offs=offs):
                for off in offs:
                    for k in range(NROW // _CH):
                        pltpu.sync_copy(z_hbm,
                                        acc.at[pl.ds(row0 + k * _CH, _CH)])
                    plsc.subcore_barrier()

                    def body(j, carry, off=off):
                        e0 = s * EP + j * _CH
                        pltpu.sync_copy(
                            vals_hbm.at[pl.ds(e0, _CH), pl.ds(off, 128)], buf)
                        pltpu.sync_copy(buf, acc.at[idxv.at[j, 0]], add=True)
                        return carry

                    lax.fori_loop(0, NCH, body, 0)
                    plsc.subcore_barrier()
                    pltpu.sync_copy(
                        acc.at[pl.ds(row0, NROW)],
                        out_hbm.at[pl.ds(row0, NROW), pl.ds(off, 128)])
                    plsc.subcore_barrier()

    return scat(vals, idx2d, zrows)[:n]


def _conv(h, src, dst1d, dst2d, zrows, pre2, trig, deg, rw, M, K, in_order):
    n = h.shape[0]
    ri = _rep_dim(in_order)
    ro, co = M.shape[2], K.shape[1]
    if in_order == 0:
        table = jnp.einsum('nri,io->nro', h, K).reshape(n, co)
    else:
        table = h.reshape(n, ri * K.shape[0])
    g = jnp.take(table, src, axis=0)
    if ro * co >= 128:
        dpad = -(ro * co) // 128 * -128
        out_msg = _edge_math(g, pre2, trig, rw.reshape(1, 2 * _B),
                             M.reshape(_B, ri * ro), K, in_order, ro, co, dpad)
        out = _sc_scatter_add(out_msg, dst2d, zrows, n)[:, :ro * co]
    else:
        out_msg = _edge_math(g, pre2, trig, rw.reshape(1, 2 * _B),
                             M.reshape(_B, ri * ro), K, in_order, ro, co,
                             ro * co)
        out = jnp.zeros((n, ro * co), jnp.float32).at[dst1d].add(out_msg)
    out = out / deg[:, None]
    return out.reshape(n, ro, co)


def _bn(h, g, b):
    mean = h.mean(axis=(0, 1), keepdims=True)
    var = h.var(axis=(0, 1), keepdims=True)
    return (h - mean) / jnp.sqrt(var + 1e-5) * g + b


def kernel(x, edge_index, precomp, connection, params):
    src, dst = edge_index[0], edge_index[1]
    n = x.shape[0]
    E = src.shape[0]
    pre2 = precomp.reshape(E, 2 * _B)
    trig = jnp.stack([jnp.cos(connection), jnp.sin(connection),
                      jnp.cos(2 * connection), jnp.sin(2 * connection)], axis=1)
    deg = jnp.clip(jnp.zeros((n,), jnp.float32).at[dst].add(1.0), 1.0)
    dst2d = dst.reshape(E // _CH, 1, _CH)
    zrows = jnp.zeros((_CH, 128), jnp.float32)
    blocks = [(0, 2, 128, 64, False), (2, 2, 64, 64, False), (2, 0, 64, 10, True)]
    h = x[:, None, :]
    for i, (oi, oo, ci, co, last) in enumerate(blocks):
        def p(name, i=i):
            return params[f"b{i}_{name}"]
        t = _conv(h, src, dst, dst2d, zrows, pre2, trig, deg, p('rw1'),
                  p('M1'), p('K1'), oi)
        t = _bn(t, p('g1'), p('be1'))
        t = jax.nn.relu(t)
        t = _conv(t, src, dst, dst2d, zrows, pre2, trig, deg, p('rw2'),
                  p('M2'), p('K2'), oo)
        t = _bn(t, p('g2'), p('be2'))
        sc = jnp.einsum('nri,rs,io->nso', h, p('S'), p('Ks'))
        h = t + sc
        if not last:
            h = jax.nn.relu(h)
    return h[:, 0, :]


# speedup vs baseline: 11.9403x; 1.3721x over previous
from functools import partial

import jax
import jax.numpy as jnp
from jax import lax
from jax.experimental import pallas as pl
from jax.experimental.pallas import tpu as pltpu
from jax.experimental.pallas import tpu_sc as plsc

_B = 5
_CH = 80
_NS = 16


def _rep_dim(order):
    return 1 if order == 0 else 2 * order + 1


def _bf(v):
    return v.astype(jnp.bfloat16).astype(jnp.float32)


def _edge_body(pre_ref, trig_ref, g_ref, rw_ref, m_ref, k_ref, out_ref, *,
               in_order, ri, ro, co, ci, dpad):
    pre = pre_ref[...]
    rw = rw_ref[...]
    coeff = (_bf(pre[:, :_B]) * _bf(rw[:, :_B])
             + _bf(pre[:, _B:]) * _bf(rw[:, _B:]))
    m2d = m_ref[...]
    T = jnp.dot(coeff.astype(jnp.bfloat16), m2d.astype(jnp.bfloat16),
                preferred_element_type=jnp.float32)
    g = g_ref[...]
    if in_order == 0:
        ys = [_bf(g[:, :co])]
    else:
        trig = trig_ref[...]
        msgs = [g[:, :ci]]
        for m in range(1, in_order + 1):
            c = trig[:, 2 * m - 2:2 * m - 1]
            s = trig[:, 2 * m - 1:2 * m]
            re = g[:, (2 * m - 1) * ci:(2 * m) * ci]
            im = g[:, (2 * m) * ci:(2 * m + 1) * ci]
            msgs.append(c * re - s * im)
            msgs.append(s * re + c * im)
        kb = k_ref[...].astype(jnp.bfloat16)
        ys = [_bf(jnp.dot(m_.astype(jnp.bfloat16), kb,
                          preferred_element_type=jnp.float32)) for m_ in msgs]
    for s in range(ro):
        acc = None
        for r in range(ri):
            term = _bf(T[:, r * ro + s:r * ro + s + 1]) * ys[r]
            acc = term if acc is None else acc + term
        out_ref[:, s * co:(s + 1) * co] = acc
    if dpad > ro * co:
        out_ref[:, ro * co:] = jnp.zeros((out_ref.shape[0], dpad - ro * co),
                                         jnp.float32)


def _edge_math(g, pre2, trig, rw2, m2d, K, in_order, ro, co, dpad, blk=512):
    E = g.shape[0]
    ri = _rep_dim(in_order)
    ci = K.shape[0]
    return pl.pallas_call(
        partial(_edge_body, in_order=in_order, ri=ri, ro=ro, co=co, ci=ci,
                dpad=dpad),
        grid=(E // blk,),
        in_specs=[
            pl.BlockSpec((blk, 2 * _B), lambda i: (i, 0)),
            pl.BlockSpec((blk, 4), lambda i: (i, 0)),
            pl.BlockSpec((blk, g.shape[1]), lambda i: (i, 0)),
            pl.BlockSpec((1, 2 * _B), lambda i: (0, 0)),
            pl.BlockSpec((_B, ri * ro), lambda i: (0, 0)),
            pl.BlockSpec((ci, co), lambda i: (0, 0)),
        ],
        out_specs=pl.BlockSpec((blk, dpad), lambda i: (i, 0)),
        out_shape=jax.ShapeDtypeStruct((E, dpad), jnp.float32),
    )(pre2, trig, g, rw2, m2d, K)


def _sc_scatter_add(vals, idx2d, zrows, n):
    E, D = vals.shape
    EP = E // _NS
    NCH = EP // _CH
    npad = _NS * ((n + _NS * _CH - 1) // (_NS * _CH)) * _CH
    NROW = npad // _NS
    G = D // 128
    group_offs = (tuple(range(0, (G + 1) // 2 * 128, 128)),
                  tuple(range((G + 1) // 2 * 128, D, 128)))

    @partial(pl.kernel,
             out_type=jax.ShapeDtypeStruct((npad, D), jnp.float32),
             mesh=plsc.VectorSubcoreMesh(core_axis_name="c",
                                         subcore_axis_name="s"),
             scratch_types=[
                 pltpu.VMEM_SHARED((npad, 128), jnp.float32),
                 pltpu.VMEM((NCH, 1, _CH), jnp.int32),
                 pltpu.VMEM((_CH, 128), jnp.float32),
             ])
    def scat(vals_hbm, idx_hbm, z_hbm, out_hbm, acc, idxv, buf):
        c = lax.axis_index("c")
        s = lax.axis_index("s")
        row0 = s * NROW
        pltpu.sync_copy(idx_hbm.at[pl.ds(s * NCH, NCH)], idxv)

        for core_id, offs in enumerate(group_offs):
            if not offs:
                continue

            @pl.when(c == core_id)
            def _(offs=offs):
                for off in offs:
                    for k in range(NROW // _CH):
                        pltpu.sync_copy(z_hbm,
                                        acc.at[pl.ds(row0 + k * _CH, _CH)])
                    plsc.subcore_barrier()

                    def body(j, carry, off=off):
                        e0 = s * EP + j * _CH
                        pltpu.sync_copy(
                            vals_hbm.at[pl.ds(e0, _CH), pl.ds(off, 128)], buf)
                        pltpu.sync_copy(buf, acc.at[idxv.at[j, 0]], add=True)
                        return carry

                    lax.fori_loop(0, NCH, body, 0)
                    plsc.subcore_barrier()
                    pltpu.sync_copy(
                        acc.at[pl.ds(row0, NROW)],
                        out_hbm.at[pl.ds(row0, NROW), pl.ds(off, 128)])
                    plsc.subcore_barrier()

    return scat(vals, idx2d, zrows)[:n]


def _sc_gather(table, idx2d, E):
    n, D = table.shape
    NW = 2 * _NS
    EP = E // NW
    NCH = EP // _CH

    @partial(pl.kernel,
             out_type=jax.ShapeDtypeStruct((E, D), jnp.float32),
             mesh=plsc.VectorSubcoreMesh(core_axis_name="c",
                                         subcore_axis_name="s"),
             scratch_types=[
                 pltpu.VMEM((NCH, 1, _CH), jnp.int32),
                 pltpu.VMEM((_CH, D), jnp.float32),
                 pltpu.VMEM((_CH, D), jnp.float32),
                 pltpu.SemaphoreType.DMA,
                 pltpu.SemaphoreType.DMA,
             ])
    def gath(table_hbm, idx_hbm, out_hbm, idxv, b0, b1, sem0, sem1):
        c = lax.axis_index("c")
        s = lax.axis_index("s")
        wid = s * 2 + c
        e0w = wid * EP
        pltpu.sync_copy(idx_hbm.at[pl.ds(wid * NCH, NCH)], idxv)

        def body(jj, carry):
            j0 = 2 * jj
            d0 = pltpu.async_copy(table_hbm.at[idxv.at[j0, 0]], b0, sem0)
            d1 = pltpu.async_copy(table_hbm.at[idxv.at[j0 + 1, 0]], b1, sem1)
            d0.wait()
            pltpu.sync_copy(b0, out_hbm.at[pl.ds(e0w + j0 * _CH, _CH)])
            d1.wait()
            pltpu.sync_copy(b1, out_hbm.at[pl.ds(e0w + (j0 + 1) * _CH, _CH)])
            return carry

        lax.fori_loop(0, NCH // 2, body, 0)
        if NCH % 2:
            j = NCH - 1
            pltpu.async_copy(table_hbm.at[idxv.at[j, 0]], b0, sem0).wait()
            pltpu.sync_copy(b0, out_hbm.at[pl.ds(e0w + j * _CH, _CH)])

    return gath(table, idx2d)


def _conv(h, src, dst1d, dst2d, zrows, pre2, trig, deg, rw, M, K, in_order):
    n = h.shape[0]
    ri = _rep_dim(in_order)
    ro, co = M.shape[2], K.shape[1]
    if in_order == 0:
        table = jnp.einsum('nri,io->nro', h, K).reshape(n, co)
    else:
        table = h.reshape(n, ri * K.shape[0])
    E = src.shape[0] * src.shape[2]
    tw = table.shape[1]
    twp = -(-tw // 128) * 128
    tpad = table if twp == tw else jnp.pad(table, ((0, 0), (0, twp - tw)))
    g = _sc_gather(tpad, src, E)
    if ro * co >= 128:
        dpad = -(ro * co) // 128 * -128
        out_msg = _edge_math(g, pre2, trig, rw.reshape(1, 2 * _B),
                             M.reshape(_B, ri * ro), K, in_order, ro, co, dpad)
        out = _sc_scatter_add(out_msg, dst2d, zrows, n)[:, :ro * co]
    else:
        out_msg = _edge_math(g, pre2, trig, rw.reshape(1, 2 * _B),
                             M.reshape(_B, ri * ro), K, in_order, ro, co,
                             ro * co)
        out = jnp.zeros((n, ro * co), jnp.float32).at[dst1d].add(out_msg)
    out = out / deg[:, None]
    return out.reshape(n, ro, co)


def _bn(h, g, b):
    mean = h.mean(axis=(0, 1), keepdims=True)
    var = h.var(axis=(0, 1), keepdims=True)
    return (h - mean) / jnp.sqrt(var + 1e-5) * g + b


def kernel(x, edge_index, precomp, connection, params):
    src, dst = edge_index[0], edge_index[1]
    n = x.shape[0]
    E = src.shape[0]
    pre2 = precomp.reshape(E, 2 * _B)
    trig = jnp.stack([jnp.cos(connection), jnp.sin(connection),
                      jnp.cos(2 * connection), jnp.sin(2 * connection)], axis=1)
    deg = jnp.clip(jnp.zeros((n,), jnp.float32).at[dst].add(1.0), 1.0)
    dst2d = dst.reshape(E // _CH, 1, _CH)
    src2d = src.reshape(E // _CH, 1, _CH)
    zrows = jnp.zeros((_CH, 128), jnp.float32)
    blocks = [(0, 2, 128, 64, False), (2, 2, 64, 64, False), (2, 0, 64, 10, True)]
    h = x[:, None, :]
    for i, (oi, oo, ci, co, last) in enumerate(blocks):
        def p(name, i=i):
            return params[f"b{i}_{name}"]
        t = _conv(h, src2d, dst, dst2d, zrows, pre2, trig, deg, p('rw1'),
                  p('M1'), p('K1'), oi)
        t = _bn(t, p('g1'), p('be1'))
        t = jax.nn.relu(t)
        t = _conv(t, src2d, dst, dst2d, zrows, pre2, trig, deg, p('rw2'),
                  p('M2'), p('K2'), oo)
        t = _bn(t, p('g2'), p('be2'))
        sc = jnp.einsum('nri,rs,io->nso', h, p('S'), p('Ks'))
        h = t + sc
        if not last:
            h = jax.nn.relu(h)
    return h[:, 0, :]


# speedup vs baseline: 12.7122x; 1.0646x over previous
from functools import partial

import jax
import jax.numpy as jnp
from jax import lax
from jax.experimental import pallas as pl
from jax.experimental.pallas import tpu as pltpu
from jax.experimental.pallas import tpu_sc as plsc

_B = 5
_CH = 80
_NS = 16


def _rep_dim(order):
    return 1 if order == 0 else 2 * order + 1


def _bf(v):
    return v.astype(jnp.bfloat16).astype(jnp.float32)


def _edge_body(pre_ref, trig_ref, g_ref, rw_ref, m_ref, k_ref, out_ref, *,
               in_order, ri, ro, co, ci, dpad):
    pre = pre_ref[...]
    rw = rw_ref[...]
    coeff = (_bf(pre[:, :_B]) * _bf(rw[:, :_B])
             + _bf(pre[:, _B:]) * _bf(rw[:, _B:]))
    m2d = m_ref[...]
    T = jnp.dot(coeff.astype(jnp.bfloat16), m2d.astype(jnp.bfloat16),
                preferred_element_type=jnp.float32)
    g = g_ref[...]
    if in_order == 0:
        ys = [_bf(g[:, :co])]
    else:
        trig = trig_ref[...]
        msgs = [g[:, :ci]]
        for m in range(1, in_order + 1):
            c = trig[:, 2 * m - 2:2 * m - 1]
            s = trig[:, 2 * m - 1:2 * m]
            re = g[:, (2 * m - 1) * ci:(2 * m) * ci]
            im = g[:, (2 * m) * ci:(2 * m + 1) * ci]
            msgs.append(c * re - s * im)
            msgs.append(s * re + c * im)
        kb = k_ref[...].astype(jnp.bfloat16)
        ys = [_bf(jnp.dot(m_.astype(jnp.bfloat16), kb,
                          preferred_element_type=jnp.float32)) for m_ in msgs]
    for s in range(ro):
        acc = None
        for r in range(ri):
            term = _bf(T[:, r * ro + s:r * ro + s + 1]) * ys[r]
            acc = term if acc is None else acc + term
        out_ref[:, s * co:(s + 1) * co] = acc
    if dpad > ro * co:
        out_ref[:, ro * co:] = jnp.zeros((out_ref.shape[0], dpad - ro * co),
                                         jnp.float32)


def _edge_math(g, pre2, trig, rw2, m2d, K, in_order, ro, co, dpad, blk=512):
    E = g.shape[0]
    ri = _rep_dim(in_order)
    ci = K.shape[0]
    return pl.pallas_call(
        partial(_edge_body, in_order=in_order, ri=ri, ro=ro, co=co, ci=ci,
                dpad=dpad),
        grid=(E // blk,),
        in_specs=[
            pl.BlockSpec((blk, 2 * _B), lambda i: (i, 0)),
            pl.BlockSpec((blk, 4), lambda i: (i, 0)),
            pl.BlockSpec((blk, g.shape[1]), lambda i: (i, 0)),
            pl.BlockSpec((1, 2 * _B), lambda i: (0, 0)),
            pl.BlockSpec((_B, ri * ro), lambda i: (0, 0)),
            pl.BlockSpec((ci, co), lambda i: (0, 0)),
        ],
        out_specs=pl.BlockSpec((blk, dpad), lambda i: (i, 0)),
        out_shape=jax.ShapeDtypeStruct((E, dpad), jnp.float32),
    )(pre2, trig, g, rw2, m2d, K)


def _sc_scatter_add(vals, idx2d, zrows, n):
    E, D = vals.shape
    EP = E // _NS
    NCH = EP // _CH
    npad = _NS * ((n + _NS * _CH - 1) // (_NS * _CH)) * _CH
    NROW = npad // _NS
    G = D // 128
    group_offs = (tuple(range(0, (G + 1) // 2 * 128, 128)),
                  tuple(range((G + 1) // 2 * 128, D, 128)))

    @partial(pl.kernel,
             out_type=jax.ShapeDtypeStruct((npad, D), jnp.float32),
             mesh=plsc.VectorSubcoreMesh(core_axis_name="c",
                                         subcore_axis_name="s"),
             scratch_types=[
                 pltpu.VMEM_SHARED((npad, 128), jnp.float32),
                 pltpu.VMEM((1, _CH), jnp.int32),
                 pltpu.VMEM((1, _CH), jnp.int32),
                 pltpu.VMEM((_CH, 128), jnp.float32),
                 pltpu.VMEM((_CH, 128), jnp.float32),
                 pltpu.SemaphoreType.DMA,
                 pltpu.SemaphoreType.DMA,
                 pltpu.SemaphoreType.DMA,
                 pltpu.SemaphoreType.DMA,
                 pltpu.SemaphoreType.DMA,
                 pltpu.SemaphoreType.DMA,
             ])
    def scat(vals_hbm, idx_hbm, z_hbm, out_hbm, acc, ib0, ib1, b0, b1,
             si0, si1, sl0, sl1, sa0, sa1):
        c = lax.axis_index("c")
        s = lax.axis_index("s")
        row0 = s * NROW
        bufs, sls, sas = (b0, b1), (sl0, sl1), (sa0, sa1)
        ibs, sis = (ib0, ib1), (si0, si1)

        for core_id, offs in enumerate(group_offs):
            if not offs:
                continue

            @pl.when(c == core_id)
            def _(offs=offs):
                for off in offs:
                    for k in range(NROW // _CH):
                        pltpu.sync_copy(z_hbm,
                                        acc.at[pl.ds(row0 + k * _CH, _CH)])
                    plsc.subcore_barrier()

                    def body(jj, carry, off=off):
                        j0 = 2 * jj
                        ils = [pltpu.async_copy(
                            idx_hbm.at[s * NCH + j0 + b], ibs[b], sis[b])
                            for b in range(2)]
                        ls = [pltpu.async_copy(
                            vals_hbm.at[pl.ds(s * EP + (j0 + b) * _CH, _CH),
                                        pl.ds(off, 128)],
                            bufs[b], sls[b]) for b in range(2)]
                        ads = []
                        for b in range(2):
                            ils[b].wait()
                            ls[b].wait()
                            ads.append(pltpu.async_copy(
                                bufs[b], acc.at[ibs[b].at[0]],
                                sas[b], add=True))
                        for a in ads:
                            a.wait()
                        return carry

                    lax.fori_loop(0, NCH // 2, body, 0)
                    for j in range(NCH // 2 * 2, NCH):
                        pltpu.sync_copy(idx_hbm.at[s * NCH + j], ib0)
                        pltpu.sync_copy(
                            vals_hbm.at[pl.ds(s * EP + j * _CH, _CH),
                                        pl.ds(off, 128)], bufs[0])
                        pltpu.sync_copy(bufs[0], acc.at[ib0.at[0]],
                                        add=True)
                    plsc.subcore_barrier()
                    pltpu.sync_copy(
                        acc.at[pl.ds(row0, NROW)],
                        out_hbm.at[pl.ds(row0, NROW), pl.ds(off, 128)])
                    plsc.subcore_barrier()

    return scat(vals, idx2d, zrows)[:n]


def _sc_gather(table, idx2d, E):
    n, D = table.shape
    NW = 2 * _NS
    EP = E // NW
    NCH = EP // _CH

    @partial(pl.kernel,
             out_type=jax.ShapeDtypeStruct((E, D), jnp.float32),
             mesh=plsc.VectorSubcoreMesh(core_axis_name="c",
                                         subcore_axis_name="s"),
             scratch_types=[
                 pltpu.VMEM((NCH, 1, _CH), jnp.int32),
                 pltpu.VMEM((_CH, D), jnp.float32),
                 pltpu.VMEM((_CH, D), jnp.float32),
                 pltpu.SemaphoreType.DMA,
                 pltpu.SemaphoreType.DMA,
                 pltpu.SemaphoreType.DMA,
                 pltpu.SemaphoreType.DMA,
             ])
    def gath(table_hbm, idx_hbm, out_hbm, idxv, b0, b1,
             sg0, sg1, ss0, ss1):
        c = lax.axis_index("c")
        s = lax.axis_index("s")
        wid = s * 2 + c
        e0w = wid * EP
        pltpu.sync_copy(idx_hbm.at[pl.ds(wid * NCH, NCH)], idxv)
        bufs, sgs, sss = (b0, b1), (sg0, sg1), (ss0, ss1)

        def body(jj, carry):
            j0 = 2 * jj
            gs = [pltpu.async_copy(table_hbm.at[idxv.at[j0 + b, 0]],
                                   bufs[b], sgs[b]) for b in range(2)]
            ts = []
            for b in range(2):
                gs[b].wait()
                ts.append(pltpu.async_copy(
                    bufs[b],
                    out_hbm.at[pl.ds(e0w + (j0 + b) * _CH, _CH)], sss[b]))
            for t in ts:
                t.wait()
            return carry

        lax.fori_loop(0, NCH // 2, body, 0)
        for j in range(NCH // 2 * 2, NCH):
            pltpu.async_copy(table_hbm.at[idxv.at[j, 0]], b0, sg0).wait()
            pltpu.sync_copy(b0, out_hbm.at[pl.ds(e0w + j * _CH, _CH)])

    return gath(table, idx2d)


def _conv(h, src, dst1d, dst2d, zrows, pre2, trig, deg, rw, M, K, in_order):
    n = h.shape[0]
    ri = _rep_dim(in_order)
    ro, co = M.shape[2], K.shape[1]
    if in_order == 0:
        table = jnp.einsum('nri,io->nro', h, K).reshape(n, co)
    else:
        table = h.reshape(n, ri * K.shape[0])
    E = src.shape[0] * src.shape[2]
    tw = table.shape[1]
    twp = -(-tw // 128) * 128
    tpad = table if twp == tw else jnp.pad(table, ((0, 0), (0, twp - tw)))
    g = _sc_gather(tpad, src, E)
    if ro * co >= 128:
        dpad = -(ro * co) // 128 * -128
        out_msg = _edge_math(g, pre2, trig, rw.reshape(1, 2 * _B),
                             M.reshape(_B, ri * ro), K, in_order, ro, co, dpad)
        out = _sc_scatter_add(out_msg, dst2d, zrows, n)[:, :ro * co]
    else:
        out_msg = _edge_math(g, pre2, trig, rw.reshape(1, 2 * _B),
                             M.reshape(_B, ri * ro), K, in_order, ro, co,
                             ro * co)
        out = jnp.zeros((n, ro * co), jnp.float32).at[dst1d].add(out_msg)
    out = out / deg[:, None]
    return out.reshape(n, ro, co)


def _bn(h, g, b):
    mean = h.mean(axis=(0, 1), keepdims=True)
    var = h.var(axis=(0, 1), keepdims=True)
    return (h - mean) / jnp.sqrt(var + 1e-5) * g + b


def kernel(x, edge_index, precomp, connection, params):
    src, dst = edge_index[0], edge_index[1]
    n = x.shape[0]
    E = src.shape[0]
    pre2 = precomp.reshape(E, 2 * _B)
    trig = jnp.stack([jnp.cos(connection), jnp.sin(connection),
                      jnp.cos(2 * connection), jnp.sin(2 * connection)], axis=1)
    deg = jnp.clip(jnp.zeros((n,), jnp.float32).at[dst].add(1.0), 1.0)
    dst2d = dst.reshape(E // _CH, 1, _CH)
    src2d = src.reshape(E // _CH, 1, _CH)
    zrows = jnp.zeros((_CH, 128), jnp.float32)
    blocks = [(0, 2, 128, 64, False), (2, 2, 64, 64, False), (2, 0, 64, 10, True)]
    h = x[:, None, :]
    for i, (oi, oo, ci, co, last) in enumerate(blocks):
        def p(name, i=i):
            return params[f"b{i}_{name}"]
        t = _conv(h, src2d, dst, dst2d, zrows, pre2, trig, deg, p('rw1'),
                  p('M1'), p('K1'), oi)
        t = _bn(t, p('g1'), p('be1'))
        t = jax.nn.relu(t)
        t = _conv(t, src2d, dst, dst2d, zrows, pre2, trig, deg, p('rw2'),
                  p('M2'), p('K2'), oo)
        t = _bn(t, p('g2'), p('be2'))
        sc = jnp.einsum('nri,rs,io->nso', h, p('S'), p('Ks'))
        h = t + sc
        if not last:
            h = jax.nn.relu(h)
    return h[:, 0, :]


# speedup vs baseline: 14.2880x; 1.1240x over previous
from functools import partial

import jax
import jax.numpy as jnp
from jax import lax
from jax.experimental import pallas as pl
from jax.experimental.pallas import tpu as pltpu
from jax.experimental.pallas import tpu_sc as plsc

_B = 5
_CH = 80
_NS = 16


def _rep_dim(order):
    return 1 if order == 0 else 2 * order + 1


def _bf(v):
    return v.astype(jnp.bfloat16).astype(jnp.float32)


def _edge_body(pre_ref, trig_ref, g_ref, rw_ref, m_ref, k_ref, out_ref, *,
               in_order, ri, ro, co, ci, dpad):
    pre = pre_ref[...]
    rw = rw_ref[...]
    coeff = (_bf(pre[:, :_B]) * _bf(rw[:, :_B])
             + _bf(pre[:, _B:]) * _bf(rw[:, _B:]))
    m2d = m_ref[...]
    T = jnp.dot(coeff.astype(jnp.bfloat16), m2d.astype(jnp.bfloat16),
                preferred_element_type=jnp.float32)
    g = g_ref[...]
    if in_order == 0:
        ys = [_bf(g[:, :co])]
    else:
        trig = trig_ref[...]
        msgs = [g[:, :ci]]
        for m in range(1, in_order + 1):
            c = trig[:, 2 * m - 2:2 * m - 1]
            s = trig[:, 2 * m - 1:2 * m]
            re = g[:, (2 * m - 1) * ci:(2 * m) * ci]
            im = g[:, (2 * m) * ci:(2 * m + 1) * ci]
            msgs.append(c * re - s * im)
            msgs.append(s * re + c * im)
        kb = k_ref[...].astype(jnp.bfloat16)
        ys = [_bf(jnp.dot(m_.astype(jnp.bfloat16), kb,
                          preferred_element_type=jnp.float32)) for m_ in msgs]
    for s in range(ro):
        acc = None
        for r in range(ri):
            term = _bf(T[:, r * ro + s:r * ro + s + 1]) * ys[r]
            acc = term if acc is None else acc + term
        out_ref[:, s * co:(s + 1) * co] = acc
    if dpad > ro * co:
        out_ref[:, ro * co:] = jnp.zeros((out_ref.shape[0], dpad - ro * co),
                                         jnp.float32)


def _edge_math(g, pre2, trig, rw2, m2d, K, in_order, ro, co, dpad, blk=512):
    E = g.shape[0]
    ri = _rep_dim(in_order)
    ci = K.shape[0]
    return pl.pallas_call(
        partial(_edge_body, in_order=in_order, ri=ri, ro=ro, co=co, ci=ci,
                dpad=dpad),
        grid=(E // blk,),
        in_specs=[
            pl.BlockSpec((blk, 2 * _B), lambda i: (i, 0)),
            pl.BlockSpec((blk, 4), lambda i: (i, 0)),
            pl.BlockSpec((blk, g.shape[1]), lambda i: (i, 0)),
            pl.BlockSpec((1, 2 * _B), lambda i: (0, 0)),
            pl.BlockSpec((_B, ri * ro), lambda i: (0, 0)),
            pl.BlockSpec((ci, co), lambda i: (0, 0)),
        ],
        out_specs=pl.BlockSpec((blk, dpad), lambda i: (i, 0)),
        out_shape=jax.ShapeDtypeStruct((E, dpad), jnp.float32),
    )(pre2, trig, g, rw2, m2d, K)


def _sc_scatter_add(vals, idx2d, zrows, n):
    E, D = vals.shape
    EP = E // _NS
    NCH = EP // _CH
    npad = _NS * ((n + _NS * _CH - 1) // (_NS * _CH)) * _CH
    NROW = npad // _NS
    G = D // 128
    H2 = NCH // 2
    whole = tuple(range(0, (G - 1) * 128, 128))
    split_off = (G - 1) * 128
    plans = (
        tuple((o, 0, NCH, o) for i, o in enumerate(whole) if i % 2 == 0)
        + ((split_off, 0, H2, split_off),),
        tuple((o, 0, NCH, o) for i, o in enumerate(whole) if i % 2 == 1)
        + ((split_off, H2, NCH, D),),
    )

    @partial(pl.kernel,
             out_type=jax.ShapeDtypeStruct((npad, D + 128), jnp.float32),
             mesh=plsc.VectorSubcoreMesh(core_axis_name="c",
                                         subcore_axis_name="s"),
             scratch_types=[
                 pltpu.VMEM_SHARED((npad, 128), jnp.float32),
                 pltpu.VMEM((1, _CH), jnp.int32),
                 pltpu.VMEM((1, _CH), jnp.int32),
                 pltpu.VMEM((_CH, 128), jnp.float32),
                 pltpu.VMEM((_CH, 128), jnp.float32),
                 pltpu.SemaphoreType.DMA,
                 pltpu.SemaphoreType.DMA,
                 pltpu.SemaphoreType.DMA,
                 pltpu.SemaphoreType.DMA,
                 pltpu.SemaphoreType.DMA,
                 pltpu.SemaphoreType.DMA,
             ])
    def scat(vals_hbm, idx_hbm, z_hbm, out_hbm, acc, ib0, ib1, b0, b1,
             si0, si1, sl0, sl1, sa0, sa1):
        c = lax.axis_index("c")
        s = lax.axis_index("s")
        row0 = s * NROW
        bufs, sls, sas = (b0, b1), (sl0, sl1), (sa0, sa1)
        ibs, sis = (ib0, ib1), (si0, si1)

        for core_id, sweeps in enumerate(plans):

            @pl.when(c == core_id)
            def _(sweeps=sweeps):
                for off, j_lo, j_hi, out_col in sweeps:
                    for k in range(NROW // _CH):
                        pltpu.sync_copy(z_hbm,
                                        acc.at[pl.ds(row0 + k * _CH, _CH)])
                    plsc.subcore_barrier()
                    nb = (j_hi - j_lo) // 2

                    def body(jj, carry, off=off, j_lo=j_lo):
                        j0 = j_lo + 2 * jj
                        ils = [pltpu.async_copy(
                            idx_hbm.at[s * NCH + j0 + b], ibs[b], sis[b])
                            for b in range(2)]
                        ls = [pltpu.async_copy(
                            vals_hbm.at[pl.ds(s * EP + (j0 + b) * _CH, _CH),
                                        pl.ds(off, 128)],
                            bufs[b], sls[b]) for b in range(2)]
                        ads = []
                        for b in range(2):
                            ils[b].wait()
                            ls[b].wait()
                            ads.append(pltpu.async_copy(
                                bufs[b], acc.at[ibs[b].at[0]],
                                sas[b], add=True))
                        for a in ads:
                            a.wait()
                        return carry

                    lax.fori_loop(0, nb, body, 0)
                    for j in range(j_lo + nb * 2, j_hi):
                        pltpu.sync_copy(idx_hbm.at[s * NCH + j], ib0)
                        pltpu.sync_copy(
                            vals_hbm.at[pl.ds(s * EP + j * _CH, _CH),
                                        pl.ds(off, 128)], bufs[0])
                        pltpu.sync_copy(bufs[0], acc.at[ib0.at[0]],
                                        add=True)
                    plsc.subcore_barrier()
                    pltpu.sync_copy(
                        acc.at[pl.ds(row0, NROW)],
                        out_hbm.at[pl.ds(row0, NROW), pl.ds(out_col, 128)])
                    plsc.subcore_barrier()

    res = scat(vals, idx2d, zrows)
    out = res[:, :D].at[:, split_off:split_off + 128].add(res[:, D:])
    return out[:n]


def _sc_gather(table, idx2d, E):
    n, D = table.shape
    NW = 2 * _NS
    EP = E // NW
    NCH = EP // _CH

    @partial(pl.kernel,
             out_type=jax.ShapeDtypeStruct((E, D), jnp.float32),
             mesh=plsc.VectorSubcoreMesh(core_axis_name="c",
                                         subcore_axis_name="s"),
             scratch_types=[
                 pltpu.VMEM((NCH, 1, _CH), jnp.int32),
                 pltpu.VMEM((_CH, D), jnp.float32),
                 pltpu.VMEM((_CH, D), jnp.float32),
                 pltpu.SemaphoreType.DMA,
                 pltpu.SemaphoreType.DMA,
                 pltpu.SemaphoreType.DMA,
                 pltpu.SemaphoreType.DMA,
             ])
    def gath(table_hbm, idx_hbm, out_hbm, idxv, b0, b1,
             sg0, sg1, ss0, ss1):
        c = lax.axis_index("c")
        s = lax.axis_index("s")
        wid = s * 2 + c
        e0w = wid * EP
        pltpu.sync_copy(idx_hbm.at[pl.ds(wid * NCH, NCH)], idxv)
        bufs, sgs, sss = (b0, b1), (sg0, sg1), (ss0, ss1)

        def body(jj, carry):
            j0 = 2 * jj
            gs = [pltpu.async_copy(table_hbm.at[idxv.at[j0 + b, 0]],
                                   bufs[b], sgs[b]) for b in range(2)]
            ts = []
            for b in range(2):
                gs[b].wait()
                ts.append(pltpu.async_copy(
                    bufs[b],
                    out_hbm.at[pl.ds(e0w + (j0 + b) * _CH, _CH)], sss[b]))
            for t in ts:
                t.wait()
            return carry

        lax.fori_loop(0, NCH // 2, body, 0)
        for j in range(NCH // 2 * 2, NCH):
            pltpu.async_copy(table_hbm.at[idxv.at[j, 0]], b0, sg0).wait()
            pltpu.sync_copy(b0, out_hbm.at[pl.ds(e0w + j * _CH, _CH)])

    return gath(table, idx2d)


def _conv(h, src, dst1d, dst2d, zrows, pre2, trig, deg, rw, M, K, in_order):
    n = h.shape[0]
    ri = _rep_dim(in_order)
    ro, co = M.shape[2], K.shape[1]
    if in_order == 0:
        table = jnp.einsum('nri,io->nro', h, K).reshape(n, co)
    else:
        table = h.reshape(n, ri * K.shape[0])
    E = src.shape[0] * src.shape[2]
    tw = table.shape[1]
    twp = -(-tw // 128) * 128
    tpad = table if twp == tw else jnp.pad(table, ((0, 0), (0, twp - tw)))
    g = _sc_gather(tpad, src, E)
    dpad = -(ro * co) // 128 * -128
    out_msg = _edge_math(g, pre2, trig, rw.reshape(1, 2 * _B),
                         M.reshape(_B, ri * ro), K, in_order, ro, co, dpad)
    out = _sc_scatter_add(out_msg, dst2d, zrows, n)[:, :ro * co]
    out = out / deg[:, None]
    return out.reshape(n, ro, co)


def _bn(h, g, b):
    mean = h.mean(axis=(0, 1), keepdims=True)
    var = h.var(axis=(0, 1), keepdims=True)
    return (h - mean) / jnp.sqrt(var + 1e-5) * g + b


def kernel(x, edge_index, precomp, connection, params):
    src, dst = edge_index[0], edge_index[1]
    n = x.shape[0]
    E = src.shape[0]
    pre2 = precomp.reshape(E, 2 * _B)
    trig = jnp.stack([jnp.cos(connection), jnp.sin(connection),
                      jnp.cos(2 * connection), jnp.sin(2 * connection)], axis=1)
    deg = jnp.clip(jnp.zeros((n,), jnp.float32).at[dst].add(1.0), 1.0)
    dst2d = dst.reshape(E // _CH, 1, _CH)
    src2d = src.reshape(E // _CH, 1, _CH)
    zrows = jnp.zeros((_CH, 128), jnp.float32)
    blocks = [(0, 2, 128, 64, False), (2, 2, 64, 64, False), (2, 0, 64, 10, True)]
    h = x[:, None, :]
    for i, (oi, oo, ci, co, last) in enumerate(blocks):
        def p(name, i=i):
            return params[f"b{i}_{name}"]
        t = _conv(h, src2d, dst, dst2d, zrows, pre2, trig, deg, p('rw1'),
                  p('M1'), p('K1'), oi)
        t = _bn(t, p('g1'), p('be1'))
        t = jax.nn.relu(t)
        t = _conv(t, src2d, dst, dst2d, zrows, pre2, trig, deg, p('rw2'),
                  p('M2'), p('K2'), oo)
        t = _bn(t, p('g2'), p('be2'))
        sc = jnp.einsum('nri,rs,io->nso', h, p('S'), p('Ks'))
        h = t + sc
        if not last:
            h = jax.nn.relu(h)
    return h[:, 0, :]


# speedup vs baseline: 15.7679x; 1.1036x over previous
from functools import partial

import jax
import jax.numpy as jnp
from jax import lax
from jax.experimental import pallas as pl
from jax.experimental.pallas import tpu as pltpu
from jax.experimental.pallas import tpu_sc as plsc

_B = 5
_CH = 80
_NS = 16


def _rep_dim(order):
    return 1 if order == 0 else 2 * order + 1


def _bf(v):
    return v.astype(jnp.bfloat16).astype(jnp.float32)


def _edge_body(pre_ref, trig_ref, g_ref, rw_ref, m_ref, k_ref, out_ref, *,
               in_order, ri, ro, co, ci, dpad, emit_deg):
    pre = pre_ref[...]
    rw = rw_ref[...]
    coeff = (_bf(pre[:, :_B]) * _bf(rw[:, :_B])
             + _bf(pre[:, _B:]) * _bf(rw[:, _B:]))
    m2d = m_ref[...]
    T = jnp.dot(coeff.astype(jnp.bfloat16), m2d.astype(jnp.bfloat16),
                preferred_element_type=jnp.float32)
    g = g_ref[...]
    if in_order == 0:
        ys = [_bf(g[:, :co])]
    else:
        trig = trig_ref[...]
        msgs = [g[:, :ci]]
        for m in range(1, in_order + 1):
            c = trig[:, 2 * m - 2:2 * m - 1]
            s = trig[:, 2 * m - 1:2 * m]
            re = g[:, (2 * m - 1) * ci:(2 * m) * ci]
            im = g[:, (2 * m) * ci:(2 * m + 1) * ci]
            msgs.append(c * re - s * im)
            msgs.append(s * re + c * im)
        kb = k_ref[...].astype(jnp.bfloat16)
        ys = [_bf(jnp.dot(m_.astype(jnp.bfloat16), kb,
                          preferred_element_type=jnp.float32)) for m_ in msgs]
    for s in range(ro):
        acc = None
        for r in range(ri):
            term = _bf(T[:, r * ro + s:r * ro + s + 1]) * ys[r]
            acc = term if acc is None else acc + term
        out_ref[:, s * co:(s + 1) * co] = acc
    if dpad > ro * co:
        if emit_deg:
            out_ref[:, ro * co:ro * co + 1] = jnp.ones(
                (out_ref.shape[0], 1), jnp.float32)
            out_ref[:, ro * co + 1:] = jnp.zeros(
                (out_ref.shape[0], dpad - ro * co - 1), jnp.float32)
        else:
            out_ref[:, ro * co:] = jnp.zeros(
                (out_ref.shape[0], dpad - ro * co), jnp.float32)


def _edge_math(g, pre2, trig, rw2, m2d, K, in_order, ro, co, dpad, emit_deg,
               blk=1024):
    E = g.shape[0]
    ri = _rep_dim(in_order)
    ci = K.shape[0]
    return pl.pallas_call(
        partial(_edge_body, in_order=in_order, ri=ri, ro=ro, co=co, ci=ci,
                dpad=dpad, emit_deg=emit_deg),
        grid=(E // blk,),
        in_specs=[
            pl.BlockSpec((blk, 2 * _B), lambda i: (i, 0)),
            pl.BlockSpec((blk, 4), lambda i: (i, 0)),
            pl.BlockSpec((blk, g.shape[1]), lambda i: (i, 0)),
            pl.BlockSpec((1, 2 * _B), lambda i: (0, 0)),
            pl.BlockSpec((_B, ri * ro), lambda i: (0, 0)),
            pl.BlockSpec((ci, co), lambda i: (0, 0)),
        ],
        out_specs=pl.BlockSpec((blk, dpad), lambda i: (i, 0)),
        out_shape=jax.ShapeDtypeStruct((E, dpad), jnp.float32),
    )(pre2, trig, g, rw2, m2d, K)


def _sc_scatter_add(vals, idx2d, zrows, n):
    E, D = vals.shape
    EP = E // _NS
    NCH = EP // _CH
    npad = _NS * ((n + _NS * _CH - 1) // (_NS * _CH)) * _CH
    NROW = npad // _NS
    G = D // 128
    H2 = NCH // 2
    whole = tuple(range(0, (G - 1) * 128, 128))
    split_off = (G - 1) * 128
    plans = (
        tuple((o, 0, NCH, o) for i, o in enumerate(whole) if i % 2 == 0)
        + ((split_off, 0, H2, split_off),),
        tuple((o, 0, NCH, o) for i, o in enumerate(whole) if i % 2 == 1)
        + ((split_off, H2, NCH, D),),
    )

    @partial(pl.kernel,
             out_type=jax.ShapeDtypeStruct((npad, D + 128), jnp.float32),
             mesh=plsc.VectorSubcoreMesh(core_axis_name="c",
                                         subcore_axis_name="s"),
             scratch_types=[
                 pltpu.VMEM_SHARED((npad, 128), jnp.float32),
                 pltpu.VMEM((1, _CH), jnp.int32),
                 pltpu.VMEM((1, _CH), jnp.int32),
                 pltpu.VMEM((_CH, 128), jnp.float32),
                 pltpu.VMEM((_CH, 128), jnp.float32),
                 pltpu.SemaphoreType.DMA,
                 pltpu.SemaphoreType.DMA,
                 pltpu.SemaphoreType.DMA,
                 pltpu.SemaphoreType.DMA,
                 pltpu.SemaphoreType.DMA,
                 pltpu.SemaphoreType.DMA,
             ])
    def scat(vals_hbm, idx_hbm, z_hbm, out_hbm, acc, ib0, ib1, b0, b1,
             si0, si1, sl0, sl1, sa0, sa1):
        c = lax.axis_index("c")
        s = lax.axis_index("s")
        row0 = s * NROW
        bufs, sls, sas = (b0, b1), (sl0, sl1), (sa0, sa1)
        ibs, sis = (ib0, ib1), (si0, si1)

        for core_id, sweeps in enumerate(plans):

            @pl.when(c == core_id)
            def _(sweeps=sweeps):
                for off, j_lo, j_hi, out_col in sweeps:
                    for k in range(NROW // _CH):
                        pltpu.sync_copy(z_hbm,
                                        acc.at[pl.ds(row0 + k * _CH, _CH)])
                    plsc.subcore_barrier()
                    nb = (j_hi - j_lo) // 2

                    def body(jj, carry, off=off, j_lo=j_lo):
                        j0 = j_lo + 2 * jj
                        ils = [pltpu.async_copy(
                            idx_hbm.at[s * NCH + j0 + b], ibs[b], sis[b])
                            for b in range(2)]
                        ls = [pltpu.async_copy(
                            vals_hbm.at[pl.ds(s * EP + (j0 + b) * _CH, _CH),
                                        pl.ds(off, 128)],
                            bufs[b], sls[b]) for b in range(2)]
                        ads = []
                        for b in range(2):
                            ils[b].wait()
                            ls[b].wait()
                            ads.append(pltpu.async_copy(
                                bufs[b], acc.at[ibs[b].at[0]],
                                sas[b], add=True))
                        for a in ads:
                            a.wait()
                        return carry

                    lax.fori_loop(0, nb, body, 0)
                    for j in range(j_lo + nb * 2, j_hi):
                        pltpu.sync_copy(idx_hbm.at[s * NCH + j], ib0)
                        pltpu.sync_copy(
                            vals_hbm.at[pl.ds(s * EP + j * _CH, _CH),
                                        pl.ds(off, 128)], bufs[0])
                        pltpu.sync_copy(bufs[0], acc.at[ib0.at[0]],
                                        add=True)
                    plsc.subcore_barrier()
                    pltpu.sync_copy(
                        acc.at[pl.ds(row0, NROW)],
                        out_hbm.at[pl.ds(row0, NROW), pl.ds(out_col, 128)])
                    plsc.subcore_barrier()

    res = scat(vals, idx2d, zrows)
    out = res[:, :D].at[:, split_off:split_off + 128].add(res[:, D:])
    return out[:n]


def _sc_gather(table, idx2d, E):
    n, D = table.shape
    NW = 2 * _NS
    EP = E // NW
    NCH = EP // _CH

    @partial(pl.kernel,
             out_type=jax.ShapeDtypeStruct((E, D), jnp.float32),
             mesh=plsc.VectorSubcoreMesh(core_axis_name="c",
                                         subcore_axis_name="s"),
             scratch_types=[
                 pltpu.VMEM((NCH, 1, _CH), jnp.int32),
                 pltpu.VMEM((_CH, D), jnp.float32),
                 pltpu.VMEM((_CH, D), jnp.float32),
                 pltpu.SemaphoreType.DMA,
                 pltpu.SemaphoreType.DMA,
                 pltpu.SemaphoreType.DMA,
                 pltpu.SemaphoreType.DMA,
             ])
    def gath(table_hbm, idx_hbm, out_hbm, idxv, b0, b1,
             sg0, sg1, ss0, ss1):
        c = lax.axis_index("c")
        s = lax.axis_index("s")
        wid = s * 2 + c
        e0w = wid * EP
        pltpu.sync_copy(idx_hbm.at[pl.ds(wid * NCH, NCH)], idxv)
        bufs, sgs, sss = (b0, b1), (sg0, sg1), (ss0, ss1)

        def body(jj, carry):
            j0 = 2 * jj
            gs = [pltpu.async_copy(table_hbm.at[idxv.at[j0 + b, 0]],
                                   bufs[b], sgs[b]) for b in range(2)]
            ts = []
            for b in range(2):
                gs[b].wait()
                ts.append(pltpu.async_copy(
                    bufs[b],
                    out_hbm.at[pl.ds(e0w + (j0 + b) * _CH, _CH)], sss[b]))
            for t in ts:
                t.wait()
            return carry

        lax.fori_loop(0, NCH // 2, body, 0)
        for j in range(NCH // 2 * 2, NCH):
            pltpu.async_copy(table_hbm.at[idxv.at[j, 0]], b0, sg0).wait()
            pltpu.sync_copy(b0, out_hbm.at[pl.ds(e0w + j * _CH, _CH)])

    return gath(table, idx2d)


def _conv(h, src, dst2d, zrows, pre2, trig, deg, rw, M, K, in_order):
    n = h.shape[0]
    ri = _rep_dim(in_order)
    ro, co = M.shape[2], K.shape[1]
    if in_order == 0:
        table = jnp.einsum('nri,io->nro', h, K).reshape(n, co)
    else:
        table = h.reshape(n, ri * K.shape[0])
    E = src.shape[0] * src.shape[2]
    tw = table.shape[1]
    twp = -(-tw // 128) * 128
    tpad = table if twp == tw else jnp.pad(table, ((0, 0), (0, twp - tw)))
    g = _sc_gather(tpad, src, E)
    dpad = -(ro * co) // 128 * -128
    emit_deg = deg is None
    out_msg = _edge_math(g, pre2, trig, rw.reshape(1, 2 * _B),
                         M.reshape(_B, ri * ro), K, in_order, ro, co, dpad,
                         emit_deg)
    outp = _sc_scatter_add(out_msg, dst2d, zrows, n)
    if emit_deg:
        deg = jnp.clip(outp[:, ro * co], 1.0)
    out = outp[:, :ro * co] / deg[:, None]
    return out.reshape(n, ro, co), deg


def _bn(h, g, b):
    mean = h.mean(axis=(0, 1), keepdims=True)
    var = h.var(axis=(0, 1), keepdims=True)
    return (h - mean) / jnp.sqrt(var + 1e-5) * g + b


def kernel(x, edge_index, precomp, connection, params):
    src, dst = edge_index[0], edge_index[1]
    n = x.shape[0]
    E = src.shape[0]
    pre2 = precomp.reshape(E, 2 * _B)
    trig = jnp.stack([jnp.cos(connection), jnp.sin(connection),
                      jnp.cos(2 * connection), jnp.sin(2 * connection)], axis=1)
    dst2d = dst.reshape(E // _CH, 1, _CH)
    src2d = src.reshape(E // _CH, 1, _CH)
    zrows = jnp.zeros((_CH, 128), jnp.float32)
    blocks = [(0, 2, 128, 64, False), (2, 2, 64, 64, False), (2, 0, 64, 10, True)]
    h = x[:, None, :]
    deg = None
    for i, (oi, oo, ci, co, last) in enumerate(blocks):
        def p(name, i=i):
            return params[f"b{i}_{name}"]
        t, deg = _conv(h, src2d, dst2d, zrows, pre2, trig, deg, p('rw1'),
                       p('M1'), p('K1'), oi)
        t = _bn(t, p('g1'), p('be1'))
        t = jax.nn.relu(t)
        t, _ = _conv(t, src2d, dst2d, zrows, pre2, trig, deg, p('rw2'),
                     p('M2'), p('K2'), oo)
        t = _bn(t, p('g2'), p('be2'))
        sc = jnp.einsum('nri,rs,io->nso', h, p('S'), p('Ks'))
        h = t + sc
        if not last:
            h = jax.nn.relu(h)
    return h[:, 0, :]
